# Initial kernel scaffold; baseline (speedup 1.0000x reference)
import jax
import jax.numpy as jnp
from jax.experimental import pallas as pl


def kernel(iq_signal, params):
    raise NotImplementedError("write your pallas kernel here")


# speedup vs baseline: 23.7361x; 23.7361x over previous
import numpy as np
import jax
import jax.numpy as jnp
from jax.experimental import pallas as pl

B = 16
L = 1024
PL = 16
PS = 4
LW = 4
H = 128
NC = 8
NL = 4
P = (L - PL) // PS + 1
NPS = P * PL


def _build_masks():
    n = np.arange(NPS)
    j = n % PL
    cnt = np.minimum(j, LW) + np.minimum(PL - 1 - j, LW)
    cnt = cnt + ((j == 0) & (n >= PL)).astype(np.int64)
    inv = 1.0 / np.maximum(cnt, 1)
    cols = []
    for d in range(1, LW + 1):
        if d == 1:
            cm = (n >= 1).astype(np.float64)
        else:
            cm = (j >= d).astype(np.float64)
        cols.append(cm * inv)
    for d in range(1, LW + 1):
        cp = (j <= PL - 1 - d).astype(np.float64)
        cols.append(cp * inv)
    return np.stack(cols, axis=1).astype(np.float32)

_MASKS = _build_masks()


def _shift_down(u, d):
    return jnp.concatenate([u[NPS - d:], u[:NPS - d]], axis=0)


def _shift_up(u, d):
    return jnp.concatenate([u[d:], u[:d]], axis=0)


def _banded_mean(u, m):
    res = m[:, 0:1] * _shift_down(u, 1) + m[:, 4:5] * _shift_up(u, 1)
    for d in range(2, LW + 1):
        res = res + m[:, d - 1:d] * _shift_down(u, d)
        res = res + m[:, 3 + d:4 + d] * _shift_up(u, d)
    return res


def _ln_relu(h, g, b):
    mu = jnp.mean(h, axis=-1, keepdims=True)
    var = jnp.mean((h - mu) ** 2, axis=-1, keepdims=True)
    y = (h - mu) * jax.lax.rsqrt(var + 1e-5) * g + b
    return jnp.maximum(y, 0.0)


def _gnn_kernel(x0_ref, m_ref, wl0_ref, wr0_ref, wl_ref, wr_ref,
                b_ref, g_ref, bb_ref, out_ref):
    x0 = x0_ref[0]
    m = m_ref[...]
    i_col = x0[:, 0:1]
    q_col = x0[:, 1:2]
    u = i_col * wl0_ref[0:1, :] + q_col * wl0_ref[1:2, :]
    v = i_col * wr0_ref[0:1, :] + q_col * wr0_ref[1:2, :]
    h = _banded_mean(u, m) + v + b_ref[0:1, :]
    x = _ln_relu(h, g_ref[0:1, :], bb_ref[0:1, :])
    for i in range(1, NL):
        u = jnp.dot(x, wl_ref[i - 1], preferred_element_type=jnp.float32)
        v = jnp.dot(x, wr_ref[i - 1], preferred_element_type=jnp.float32)
        h = _banded_mean(u, m) + v + b_ref[i:i + 1, :]
        x = x + _ln_relu(h, g_ref[i:i + 1, :], bb_ref[i:i + 1, :])
    out_ref[0] = jnp.mean(x, axis=0, keepdims=True)


def _cls_kernel(sig_ref, w1_ref, b1_ref, g_ref, be_ref, w2_ref, b2_ref,
                out_ref):
    h = jnp.dot(sig_ref[...], w1_ref[...],
                preferred_element_type=jnp.float32) + b1_ref[...]
    h = _ln_relu(h, g_ref[...], be_ref[...])
    out_ref[...] = jnp.dot(h, w2_ref[...],
                           preferred_element_type=jnp.float32) + b2_ref[...]


def kernel(iq_signal, params):
    tidx = (np.arange(P)[:, None] * PS + np.arange(PL)[None, :]).reshape(-1)
    x0 = jnp.transpose(iq_signal[:, :, tidx], (0, 2, 1))

    masks = jnp.asarray(_MASKS)
    wl0 = params['sage_Wl_0']
    wr0 = params['sage_Wr_0']
    wl = jnp.stack([params['sage_Wl_%d' % i] for i in range(1, NL)])
    wr = jnp.stack([params['sage_Wr_%d' % i] for i in range(1, NL)])
    b_all = jnp.stack([params['sage_b_%d' % i] for i in range(NL)])
    g_all = jnp.stack([params['ln_g_%d' % i] for i in range(NL)])
    bb_all = jnp.stack([params['ln_b_%d' % i] for i in range(NL)])

    sig = pl.pallas_call(
        _gnn_kernel,
        grid=(B,),
        in_specs=[
            pl.BlockSpec((1, NPS, 2), lambda b: (b, 0, 0)),
            pl.BlockSpec((NPS, 8), lambda b: (0, 0)),
            pl.BlockSpec((2, H), lambda b: (0, 0)),
            pl.BlockSpec((2, H), lambda b: (0, 0)),
            pl.BlockSpec((NL - 1, H, H), lambda b: (0, 0, 0)),
            pl.BlockSpec((NL - 1, H, H), lambda b: (0, 0, 0)),
            pl.BlockSpec((NL, H), lambda b: (0, 0)),
            pl.BlockSpec((NL, H), lambda b: (0, 0)),
            pl.BlockSpec((NL, H), lambda b: (0, 0)),
        ],
        out_specs=pl.BlockSpec((1, 1, H), lambda b: (b, 0, 0)),
        out_shape=jax.ShapeDtypeStruct((B, 1, H), jnp.float32),
    )(x0, masks, wl0, wr0, wl, wr, b_all, g_all, bb_all)
    sig = sig.reshape(B, H)

    logits = pl.pallas_call(
        _cls_kernel,
        in_specs=[
            pl.BlockSpec((B, H), lambda: (0, 0)),
            pl.BlockSpec((H, H), lambda: (0, 0)),
            pl.BlockSpec((1, H), lambda: (0, 0)),
            pl.BlockSpec((1, H), lambda: (0, 0)),
            pl.BlockSpec((1, H), lambda: (0, 0)),
            pl.BlockSpec((H, NC), lambda: (0, 0)),
            pl.BlockSpec((1, NC), lambda: (0, 0)),
        ],
        out_specs=pl.BlockSpec((B, NC), lambda: (0, 0)),
        out_shape=jax.ShapeDtypeStruct((B, NC), jnp.float32),
    )(sig, params['cls_W1'], params['cls_b1'][None, :],
      params['cls_g'][None, :], params['cls_be'][None, :],
      params['cls_W2'], params['cls_b2'][None, :])
    return logits


# speedup vs baseline: 24.5537x; 1.0344x over previous
import numpy as np
import jax
import jax.numpy as jnp
from jax.experimental import pallas as pl

B = 16
L = 1024
PL = 16
PS = 4
LW = 4
H = 128
NC = 8
NL = 4
P = (L - PL) // PS + 1
NPS = P * PL


def _build_masks():
    n = np.arange(NPS)
    j = n % PL
    cnt = np.minimum(j, LW) + np.minimum(PL - 1 - j, LW)
    cnt = cnt + ((j == 0) & (n >= PL)).astype(np.int64)
    inv = 1.0 / np.maximum(cnt, 1)
    cols = []
    for d in range(1, LW + 1):
        if d == 1:
            cm = (n >= 1).astype(np.float64)
        else:
            cm = (j >= d).astype(np.float64)
        cols.append(cm * inv)
    for d in range(1, LW + 1):
        cp = (j <= PL - 1 - d).astype(np.float64)
        cols.append(cp * inv)
    return np.stack(cols, axis=1).astype(np.float32)

_MASKS = _build_masks()


def _shift_down(u, d):
    return jnp.concatenate([u[NPS - d:], u[:NPS - d]], axis=0)


def _shift_up(u, d):
    return jnp.concatenate([u[d:], u[:d]], axis=0)


def _banded_mean(u, m):
    res = m[:, 0:1] * _shift_down(u, 1) + m[:, 4:5] * _shift_up(u, 1)
    for d in range(2, LW + 1):
        res = res + m[:, d - 1:d] * _shift_down(u, d)
        res = res + m[:, 3 + d:4 + d] * _shift_up(u, d)
    return res


def _ln_relu(h, g, b):
    mu = jnp.mean(h, axis=-1, keepdims=True)
    var = jnp.mean((h - mu) ** 2, axis=-1, keepdims=True)
    y = (h - mu) * jax.lax.rsqrt(var + 1e-5) * g + b
    return jnp.maximum(y, 0.0)


def _gnn_kernel(x0_ref, m_ref, wl0_ref, wr0_ref, wlr_ref,
                b_ref, g_ref, bb_ref, out_ref):
    x0 = x0_ref[0]
    m = m_ref[...]
    i_col = x0[:, 0:1]
    q_col = x0[:, 1:2]
    u = i_col * wl0_ref[0:1, :] + q_col * wl0_ref[1:2, :]
    v = i_col * wr0_ref[0:1, :] + q_col * wr0_ref[1:2, :]
    h = _banded_mean(u, m) + v + b_ref[0:1, :]
    x = _ln_relu(h, g_ref[0:1, :], bb_ref[0:1, :])
    for i in range(1, NL):
        uv = jnp.dot(x, wlr_ref[i - 1], preferred_element_type=jnp.float32)
        h = _banded_mean(uv[:, :H], m) + uv[:, H:] + b_ref[i:i + 1, :]
        x = x + _ln_relu(h, g_ref[i:i + 1, :], bb_ref[i:i + 1, :])
    out_ref[0] = jnp.mean(x, axis=0, keepdims=True)


def _cls_kernel(sig_ref, w1_ref, b1_ref, g_ref, be_ref, w2_ref, b2_ref,
                out_ref):
    h = jnp.dot(sig_ref[...], w1_ref[...],
                preferred_element_type=jnp.float32) + b1_ref[...]
    h = _ln_relu(h, g_ref[...], be_ref[...])
    out_ref[...] = jnp.dot(h, w2_ref[...],
                           preferred_element_type=jnp.float32) + b2_ref[...]


def kernel(iq_signal, params):
    tidx = (np.arange(P)[:, None] * PS + np.arange(PL)[None, :]).reshape(-1)
    x0 = jnp.transpose(iq_signal[:, :, tidx], (0, 2, 1))

    masks = jnp.asarray(_MASKS)
    wl0 = params['sage_Wl_0']
    wr0 = params['sage_Wr_0']
    wlr = jnp.stack([
        jnp.concatenate([params['sage_Wl_%d' % i], params['sage_Wr_%d' % i]],
                        axis=1) for i in range(1, NL)])
    b_all = jnp.stack([params['sage_b_%d' % i] for i in range(NL)])
    g_all = jnp.stack([params['ln_g_%d' % i] for i in range(NL)])
    bb_all = jnp.stack([params['ln_b_%d' % i] for i in range(NL)])

    sig = pl.pallas_call(
        _gnn_kernel,
        grid=(B,),
        in_specs=[
            pl.BlockSpec((1, NPS, 2), lambda b: (b, 0, 0)),
            pl.BlockSpec((NPS, 8), lambda b: (0, 0)),
            pl.BlockSpec((2, H), lambda b: (0, 0)),
            pl.BlockSpec((2, H), lambda b: (0, 0)),
            pl.BlockSpec((NL - 1, H, 2 * H), lambda b: (0, 0, 0)),
            pl.BlockSpec((NL, H), lambda b: (0, 0)),
            pl.BlockSpec((NL, H), lambda b: (0, 0)),
            pl.BlockSpec((NL, H), lambda b: (0, 0)),
        ],
        out_specs=pl.BlockSpec((1, 1, H), lambda b: (b, 0, 0)),
        out_shape=jax.ShapeDtypeStruct((B, 1, H), jnp.float32),
    )(x0, masks, wl0, wr0, wlr, b_all, g_all, bb_all)
    sig = sig.reshape(B, H)

    logits = pl.pallas_call(
        _cls_kernel,
        in_specs=[
            pl.BlockSpec((B, H), lambda: (0, 0)),
            pl.BlockSpec((H, H), lambda: (0, 0)),
            pl.BlockSpec((1, H), lambda: (0, 0)),
            pl.BlockSpec((1, H), lambda: (0, 0)),
            pl.BlockSpec((1, H), lambda: (0, 0)),
            pl.BlockSpec((H, NC), lambda: (0, 0)),
            pl.BlockSpec((1, NC), lambda: (0, 0)),
        ],
        out_specs=pl.BlockSpec((B, NC), lambda: (0, 0)),
        out_shape=jax.ShapeDtypeStruct((B, NC), jnp.float32),
    )(sig, params['cls_W1'], params['cls_b1'][None, :],
      params['cls_g'][None, :], params['cls_be'][None, :],
      params['cls_W2'], params['cls_b2'][None, :])
    return logits
